# baseline (speedup 1.0000x reference)
import functools

import jax
import jax.numpy as jnp
from jax import lax
from jax.experimental import pallas as pl
from jax.experimental.pallas import tpu as pltpu
from jax.experimental.pallas import tpu_sc as plsc

_NC = 2
_NS = 16
_NW = _NC * _NS


def _proj_body(x_ref, w_ref, o1_ref, o2_ref):
    p = jnp.dot(x_ref[...], w_ref[...], preferred_element_type=jnp.float32)
    o1_ref[...] = p[:, :16]
    o2_ref[...] = p[:, 16:]


def _edge_body(a_ref, g_ref, w_ref, b_ref, o_ref):
    o_ref[...] = (
        jnp.dot(a_ref[...], w_ref[...], preferred_element_type=jnp.float32)
        + b_ref[...]
        + g_ref[...]
    )


def _sc_gather_sum(p1, p2, sidx, ridx, ea):
    E = sidx.shape[0]
    Do = 16
    epw = E // _NW
    C = 1000
    CP = C // 8
    nchunk = epw // C
    mesh = plsc.VectorSubcoreMesh(core_axis_name="c", subcore_axis_name="s")

    @functools.partial(
        pl.kernel,
        mesh=mesh,
        compiler_params=pltpu.CompilerParams(use_tc_tiling_on_sc=False),
        out_type=[
            jax.ShapeDtypeStruct((E // 8, 128), jnp.float32),
            jax.ShapeDtypeStruct((E // 8, 128), jnp.float32),
        ],
        scratch_types=[
            pltpu.VMEM((C,), jnp.int32),
            pltpu.VMEM((C,), jnp.int32),
            pltpu.VMEM((C, Do), jnp.float32),
            pltpu.VMEM((C, Do), jnp.float32),
            pltpu.VMEM((C, Do), jnp.float32),
            pltpu.VMEM((CP, 128), jnp.float32),
            pltpu.VMEM((CP, 128), jnp.float32),
            pltpu.SemaphoreType.DMA,
            pltpu.SemaphoreType.DMA,
        ],
    )
    def k(p1_hbm, p2_hbm, s_hbm, r_hbm, ea_hbm, out_hbm, eap_hbm,
          sidx_v, ridx_v, rows1_v, rows2_v, ea_v, out_v, eap_v, sem1, sem2):
        wid = lax.axis_index("s") * _NC + lax.axis_index("c")
        base = wid * epw

        def chunk(kk, carry):
            off = base + kk * C
            poff = off // 8
            pltpu.sync_copy(s_hbm.at[pl.ds(off, C)], sidx_v)
            pltpu.sync_copy(r_hbm.at[pl.ds(off, C)], ridx_v)
            cp1 = pltpu.async_copy(p1_hbm.at[sidx_v], rows1_v, sem1)
            cp2 = pltpu.async_copy(p2_hbm.at[ridx_v], rows2_v, sem2)
            pltpu.sync_copy(ea_hbm.at[pl.ds(off, C)], ea_v)
            cp1.wait()
            cp2.wait()

            def blk(jj, c2):
                i = jj * 8
                for t in range(8):
                    out_v[jj, 16 * t:16 * (t + 1)] = (
                        rows1_v[i + t, :] + rows2_v[i + t, :]
                    )
                    eap_v[jj, 16 * t:16 * (t + 1)] = ea_v[i + t, :]
                return c2

            lax.fori_loop(0, CP, blk, 0)
            pltpu.sync_copy(out_v, out_hbm.at[pl.ds(poff, CP)])
            pltpu.sync_copy(eap_v, eap_hbm.at[pl.ds(poff, CP)])
            return carry

        lax.fori_loop(0, nchunk, chunk, 0)

    return k(p1, p2, sidx, ridx, ea)


def kernel(x, edge_index, edge_attr, W, b):
    N, D = x.shape
    E = edge_index.shape[1]
    Do = W.shape[1]

    W1 = W[:D]
    W2 = W[D:2 * D]
    W3 = W[2 * D:]
    Wn = jnp.concatenate([W1, W2], axis=1)

    p1, p2 = pl.pallas_call(
        _proj_body,
        out_shape=[
            jax.ShapeDtypeStruct((N, Do), jnp.float32),
            jax.ShapeDtypeStruct((N, Do), jnp.float32),
        ],
    )(x, Wn)

    pack = 128 // Do
    EP = E // pack
    w3_big = jnp.kron(jnp.eye(pack, dtype=W.dtype), W3)
    b_big = jnp.tile(b, pack).reshape(1, 128)

    gp, ea_p = _sc_gather_sum(p1, p2, edge_index[0], edge_index[1], edge_attr)
    BE = 5000
    outp = pl.pallas_call(
        _edge_body,
        grid=(EP // BE,),
        in_specs=[
            pl.BlockSpec((BE, 128), lambda i: (i, 0)),
            pl.BlockSpec((BE, 128), lambda i: (i, 0)),
            pl.BlockSpec((128, 128), lambda i: (0, 0)),
            pl.BlockSpec((1, 128), lambda i: (0, 0)),
        ],
        out_specs=pl.BlockSpec((BE, 128), lambda i: (i, 0)),
        out_shape=jax.ShapeDtypeStruct((EP, 128), jnp.float32),
    )(ea_p, gp, w3_big, b_big)
    return outp.reshape(E, Do)


# speedup vs baseline: 1.1760x; 1.1760x over previous
import functools

import jax
import jax.numpy as jnp
from jax import lax
from jax.experimental import pallas as pl
from jax.experimental.pallas import tpu as pltpu
from jax.experimental.pallas import tpu_sc as plsc

_NC = 2
_NS = 16
_NW = _NC * _NS


def _proj_body(x_ref, w_ref, o1_ref, o2_ref):
    p = jnp.dot(x_ref[...], w_ref[...], preferred_element_type=jnp.float32)
    o1_ref[...] = p[:, :16]
    o2_ref[...] = p[:, 16:]


def _edge_body(a_ref, g_ref, w_ref, b_ref, o_ref):
    o_ref[...] = (
        jnp.dot(a_ref[...], w_ref[...], preferred_element_type=jnp.float32)
        + b_ref[...]
        + g_ref[...]
    )


def _sc_gather_sum(p1, p2, sidx, ridx):
    E = sidx.shape[0]
    Do = 16
    epw = E // _NW
    C = 1000
    CP = C // 8
    nchunk = epw // C
    mesh = plsc.VectorSubcoreMesh(core_axis_name="c", subcore_axis_name="s")

    @functools.partial(
        pl.kernel,
        mesh=mesh,
        compiler_params=pltpu.CompilerParams(use_tc_tiling_on_sc=False),
        out_type=jax.ShapeDtypeStruct((E // 8, 128), jnp.float32),
        scratch_types=[
            pltpu.VMEM((C,), jnp.int32),
            pltpu.VMEM((C,), jnp.int32),
            pltpu.VMEM((C, Do), jnp.float32),
            pltpu.VMEM((C, Do), jnp.float32),
            pltpu.VMEM((CP, 128), jnp.float32),
            pltpu.SemaphoreType.DMA,
            pltpu.SemaphoreType.DMA,
        ],
    )
    def k(p1_hbm, p2_hbm, s_hbm, r_hbm, out_hbm,
          sidx_v, ridx_v, rows1_v, rows2_v, out_v, sem1, sem2):
        wid = lax.axis_index("s") * _NC + lax.axis_index("c")
        base = wid * epw

        def chunk(kk, carry):
            off = base + kk * C
            poff = off // 8
            pltpu.sync_copy(s_hbm.at[pl.ds(off, C)], sidx_v)
            pltpu.sync_copy(r_hbm.at[pl.ds(off, C)], ridx_v)
            cp1 = pltpu.async_copy(p1_hbm.at[sidx_v], rows1_v, sem1)
            cp2 = pltpu.async_copy(p2_hbm.at[ridx_v], rows2_v, sem2)
            cp1.wait()
            cp2.wait()

            def blk(jj, c2):
                i = jj * 8
                for t in range(8):
                    out_v[jj, 16 * t:16 * (t + 1)] = (
                        rows1_v[i + t, :] + rows2_v[i + t, :]
                    )
                return c2

            lax.fori_loop(0, CP, blk, 0)
            pltpu.sync_copy(out_v, out_hbm.at[pl.ds(poff, CP)])
            return carry

        lax.fori_loop(0, nchunk, chunk, 0)

    return k(p1, p2, sidx, ridx)


def kernel(x, edge_index, edge_attr, W, b):
    N, D = x.shape
    E = edge_index.shape[1]
    Do = W.shape[1]

    W1 = W[:D]
    W2 = W[D:2 * D]
    W3 = W[2 * D:]
    Wn = jnp.concatenate([W1, W2], axis=1)

    p1, p2 = pl.pallas_call(
        _proj_body,
        out_shape=[
            jax.ShapeDtypeStruct((N, Do), jnp.float32),
            jax.ShapeDtypeStruct((N, Do), jnp.float32),
        ],
    )(x, Wn)

    pack = 128 // Do
    EP = E // pack
    w3_big = jnp.kron(jnp.eye(pack, dtype=W.dtype), W3)
    b_big = jnp.tile(b, pack).reshape(1, 128)

    gp = _sc_gather_sum(p1, p2, edge_index[0], edge_index[1])
    ea_p = edge_attr.reshape(EP, 128)
    BE = 5000
    outp = pl.pallas_call(
        _edge_body,
        grid=(EP // BE,),
        in_specs=[
            pl.BlockSpec((BE, 128), lambda i: (i, 0)),
            pl.BlockSpec((BE, 128), lambda i: (i, 0)),
            pl.BlockSpec((128, 128), lambda i: (0, 0)),
            pl.BlockSpec((1, 128), lambda i: (0, 0)),
        ],
        out_specs=pl.BlockSpec((BE, 128), lambda i: (i, 0)),
        out_shape=jax.ShapeDtypeStruct((EP, 128), jnp.float32),
    )(ea_p, gp, w3_big, b_big)
    return outp.reshape(E, Do)
